# baseline (speedup 1.0000x reference)
import functools

import jax
import jax.numpy as jnp
from jax import lax
from jax.experimental import pallas as pl
from jax.experimental.pallas import tpu as pltpu
from jax.experimental.pallas import tpu_sc as plsc

N = 10000
E = 320000
D_IN = 128
D_EDGE = 16
H = 128
L = 3

NC = 2
NS = 16
NW = NC * NS
EPW = E // NW
C = 40
ITER = EPW // C
GI = 25
NG = ITER // GI
NPAD = 10240
RPS = NPAD // NS


def _ln(h, g, b):
    m = jnp.mean(h, axis=-1, keepdims=True)
    v = jnp.mean((h - m) ** 2, axis=-1, keepdims=True)
    return (h - m) * jax.lax.rsqrt(v + 1e-5) * g + b


def _node_enc_body(x_ref, w1, b1, w2, b2, g, be, o_ref):
    h = jnp.maximum(jnp.dot(x_ref[...], w1[...], preferred_element_type=jnp.float32) + b1[...], 0.0)
    h = jnp.dot(h, w2[...], preferred_element_type=jnp.float32) + b2[...]
    o_ref[...] = _ln(h, g[...], be[...])


def _edge_enc_body(ea_ref, w1, b1, w2, b2, g, be, wl0, bl0, wl1, bl1, wl2, bl2,
                   o0_ref, o1_ref, o2_ref):
    h = jnp.maximum(jnp.dot(ea_ref[...], w1[...], preferred_element_type=jnp.float32) + b1[...], 0.0)
    h = jnp.dot(h, w2[...], preferred_element_type=jnp.float32) + b2[...]
    ea = _ln(h, g[...], be[...])
    o0_ref[...] = jnp.dot(ea, wl0[...], preferred_element_type=jnp.float32) + bl0[...]
    o1_ref[...] = jnp.dot(ea, wl1[...], preferred_element_type=jnp.float32) + bl1[...]
    o2_ref[...] = jnp.dot(ea, wl2[...], preferred_element_type=jnp.float32) + bl2[...]


def _node_update_body(x_ref, agg_ref, wm1, bm1, wm2, bm2, g, be, o_ref):
    h = x_ref[...] + agg_ref[0] + agg_ref[1]
    h = jnp.maximum(jnp.dot(h, wm1[...], preferred_element_type=jnp.float32) + bm1[...], 0.0)
    h = jnp.dot(h, wm2[...], preferred_element_type=jnp.float32) + bm2[...]
    o_ref[...] = jnp.maximum(_ln(h, g[...], be[...]), 0.0)


def _heads_body(x_ref, w1, b1, w2, b2, o_ref):
    h = jnp.maximum(jnp.dot(x_ref[...], w1[...], preferred_element_type=jnp.float32) + b1[...], 0.0)
    o = jnp.dot(h, w2[...], preferred_element_type=jnp.float32) + b2[...]
    sp = jnp.maximum(o, 0.0) + jnp.log(1.0 + jnp.exp(-jnp.abs(o)))
    col = lax.broadcasted_iota(jnp.int32, o.shape, 1)
    o_ref[...] = jnp.where(col == 4, sp, o)


def _sc_layer_body(x_hbm, lin_hbm, src_hbm, dst_hbm, out_hbm,
                   srcv, dstv, linA, linB, rowsA, rowsB, aggsh, semA, semB, semS):
    cid = lax.axis_index("c")
    sid = lax.axis_index("s")
    wid = cid * NS + sid

    def zrow(i, carry):
        for k in range(H // 16):
            rowsA[i, pl.ds(k * 16, 16)] = jnp.zeros((16,), jnp.float32)
        return carry
    lax.fori_loop(0, C, zrow, 0)
    for k in range(RPS // C):
        pltpu.sync_copy(rowsA, aggsh.at[pl.ds(sid * RPS + k * C, C)])
    plsc.subcore_barrier()

    def compute(rows, lin):
        def crow(i, c2):
            for r in range(2):
                for k in range(H // 16):
                    sl = (2 * i + r, pl.ds(k * 16, 16))
                    rows[sl] = jnp.maximum(rows[sl] + lin[sl], 0.0)
            return c2
        lax.fori_loop(0, C // 2, crow, 0)

    def group(g, carry):
        pltpu.sync_copy(src_hbm.at[wid * NG + g], srcv)
        pltpu.sync_copy(dst_hbm.at[wid * NG + g], dstv)
        base = g * GI * C

        def lin_cp(j, buf, sem):
            return pltpu.make_async_copy(
                lin_hbm.at[wid, pl.ds(base + j * C, C)], buf, sem)

        def gat_cp(j, buf, sem):
            return pltpu.make_async_copy(x_hbm.at[srcv.at[j]], buf, sem)

        lin_cp(0, linA, semA).start()
        gat_cp(0, rowsA, semA).start()

        def pair(k, c1):
            j0 = 2 * k
            j1 = j0 + 1
            lin_cp(j1, linB, semB).start()
            gat_cp(j1, rowsB, semB).start()
            lin_cp(j0, linA, semA).wait()
            gat_cp(j0, rowsA, semA).wait()
            compute(rowsA, linA)
            pltpu.async_copy(rowsA, aggsh.at[dstv.at[j0]], semS, add=True)
            lin_cp(j1, linB, semB).wait()
            gat_cp(j1, rowsB, semB).wait()
            compute(rowsB, linB)
            pltpu.make_async_copy(rowsA, aggsh.at[dstv.at[j0]], semS).wait()
            lin_cp(j0 + 2, linA, semA).start()
            gat_cp(j0 + 2, rowsA, semA).start()
            pltpu.sync_copy(rowsB, aggsh.at[dstv.at[j1]], add=True)
            return c1
        lax.fori_loop(0, (GI - 1) // 2, pair, 0)

        lin_cp(GI - 1, linA, semA).wait()
        gat_cp(GI - 1, rowsA, semA).wait()
        compute(rowsA, linA)
        pltpu.sync_copy(rowsA, aggsh.at[dstv.at[GI - 1]], add=True)
        return carry
    lax.fori_loop(0, NG, group, 0)

    plsc.subcore_barrier()
    pltpu.sync_copy(aggsh.at[pl.ds(sid * RPS, RPS)],
                    out_hbm.at[cid, pl.ds(sid * RPS, RPS)])


_sc_layer_cache = []


def _get_sc_layer():
    if not _sc_layer_cache:
        _sc_layer_cache.append(functools.partial(
            pl.kernel,
            out_type=jax.ShapeDtypeStruct((NC, NPAD, H), jnp.float32),
            mesh=plsc.VectorSubcoreMesh(core_axis_name="c", subcore_axis_name="s",
                                        num_cores=NC, num_subcores=NS),
            scratch_types=[
                pltpu.VMEM((GI, C), jnp.int32),
                pltpu.VMEM((GI, C), jnp.int32),
                pltpu.VMEM((C, H), jnp.float32),
                pltpu.VMEM((C, H), jnp.float32),
                pltpu.VMEM((C, H), jnp.float32),
                pltpu.VMEM((C, H), jnp.float32),
                pltpu.VMEM_SHARED((NPAD, H), jnp.float32),
                pltpu.SemaphoreType.DMA,
                pltpu.SemaphoreType.DMA,
                pltpu.SemaphoreType.DMA,
            ],
        )(_sc_layer_body))
    return _sc_layer_cache[0]


def _row_specs(nblk, rows, width):
    return pl.BlockSpec((rows, width), lambda i: (i, 0))


def _full(shape):
    return pl.BlockSpec(shape, lambda i: tuple(0 for _ in shape))


def kernel(x, edge_index, edge_attr, ne_W1, ne_b1, ne_W2, ne_b2, ne_g, ne_be, ee_W1, ee_b1, ee_W2, ee_b2, ee_g, ee_be, c0_Wl, c0_bl, c0_Wm1, c0_bm1, c0_Wm2, c0_bm2, c0_g, c0_be, c1_Wl, c1_bl, c1_Wm1, c1_bm1, c1_Wm2, c1_bm2, c1_g, c1_be, c2_Wl, c2_bl, c2_Wm1, c2_bm1, c2_Wm2, c2_bm2, c2_g, c2_be, hu_W1, hu_b1, hu_W2, hu_b2, hs_W1, hs_b1, hs_W2, hs_b2, hf_W1, hf_b1, hf_W2, hf_b2):
    f32 = jnp.float32
    r2 = lambda v: v.reshape(1, -1)

    BN = 1000
    GN = N // BN

    x0 = pl.pallas_call(
        _node_enc_body,
        grid=(GN,),
        in_specs=[_row_specs(GN, BN, D_IN), _full((D_IN, H)), _full((1, H)),
                  _full((H, H)), _full((1, H)), _full((1, H)), _full((1, H))],
        out_specs=_row_specs(GN, BN, H),
        out_shape=jax.ShapeDtypeStruct((N, H), f32),
    )(x, ne_W1, r2(ne_b1), ne_W2, r2(ne_b2), r2(ne_g), r2(ne_be))

    BE = 2000
    GE = E // BE
    lin0, lin1, lin2 = pl.pallas_call(
        _edge_enc_body,
        grid=(GE,),
        in_specs=[_row_specs(GE, BE, D_EDGE), _full((D_EDGE, H)), _full((1, H)),
                  _full((H, H)), _full((1, H)), _full((1, H)), _full((1, H)),
                  _full((H, H)), _full((1, H)), _full((H, H)), _full((1, H)),
                  _full((H, H)), _full((1, H))],
        out_specs=[_row_specs(GE, BE, H)] * 3,
        out_shape=[jax.ShapeDtypeStruct((E, H), f32)] * 3,
    )(edge_attr, ee_W1, r2(ee_b1), ee_W2, r2(ee_b2), r2(ee_g), r2(ee_be),
      c0_Wl, r2(c0_bl), c1_Wl, r2(c1_bl), c2_Wl, r2(c2_bl))

    src3d = edge_index[0].reshape(NW * NG, GI, C)
    dst3d = edge_index[1].reshape(NW * NG, GI, C)

    layer_w = [
        (lin0, c0_Wm1, c0_bm1, c0_Wm2, c0_bm2, c0_g, c0_be),
        (lin1, c1_Wm1, c1_bm1, c1_Wm2, c1_bm2, c1_g, c1_be),
        (lin2, c2_Wm1, c2_bm1, c2_Wm2, c2_bm2, c2_g, c2_be),
    ]

    xi = x0
    sc_layer = _get_sc_layer()
    for lin, wm1, bm1, wm2, bm2, g, be in layer_w:
        parts = sc_layer(xi, lin.reshape(NW, EPW, H), src3d, dst3d)
        parts = parts[:, :N]
        xi = pl.pallas_call(
            _node_update_body,
            grid=(GN,),
            in_specs=[_row_specs(GN, BN, H),
                      pl.BlockSpec((NC, BN, H), lambda i: (0, i, 0)),
                      _full((H, H)), _full((1, H)), _full((H, H)), _full((1, H)),
                      _full((1, H)), _full((1, H))],
            out_specs=_row_specs(GN, BN, H),
            out_shape=jax.ShapeDtypeStruct((N, H), f32),
        )(xi, parts, wm1, r2(bm1), wm2, r2(bm2), r2(g), r2(be))

    W1c = jnp.concatenate([hu_W1, hs_W1, hf_W1], axis=1)
    b1c = jnp.concatenate([hu_b1, hs_b1, hf_b1]).reshape(1, -1)
    W2c = jnp.zeros((192, 8), f32)
    W2c = W2c.at[0:64, 0:3].set(hu_W2)
    W2c = W2c.at[64:128, 3:4].set(hs_W2)
    W2c = W2c.at[128:192, 4:5].set(hf_W2)
    b2c = jnp.zeros((8,), f32)
    b2c = b2c.at[0:3].set(hu_b2).at[3:4].set(hs_b2).at[4:5].set(hf_b2)
    b2c = b2c.reshape(1, -1)

    heads = pl.pallas_call(
        _heads_body,
        grid=(GN,),
        in_specs=[_row_specs(GN, BN, H), _full((H, 192)), _full((1, 192)),
                  _full((192, 8)), _full((1, 8))],
        out_specs=_row_specs(GN, BN, 8),
        out_shape=jax.ShapeDtypeStruct((N, 8), f32),
    )(xi, W1c, b1c, W2c, b2c)

    return {
        "displacement": heads[:, 0:3],
        "stress": heads[:, 3:4],
        "safety_factor": heads[:, 4:5],
    }


# speedup vs baseline: 1.0688x; 1.0688x over previous
import functools

import jax
import jax.numpy as jnp
from jax import lax
from jax.experimental import pallas as pl
from jax.experimental.pallas import tpu as pltpu
from jax.experimental.pallas import tpu_sc as plsc

N = 10000
E = 320000
D_IN = 128
D_EDGE = 16
H = 128
L = 3

NC = 2
NS = 16
NW = NC * NS
EPW = E // NW
C = 40
ITER = EPW // C
GI = 25
NG = ITER // GI
NPAD = 10240
RPS = NPAD // NS


def _ln(h, g, b):
    m = jnp.mean(h, axis=-1, keepdims=True)
    v = jnp.mean((h - m) ** 2, axis=-1, keepdims=True)
    return (h - m) * jax.lax.rsqrt(v + 1e-5) * g + b


def _node_enc_body(x_ref, w1, b1, w2, b2, g, be, o_ref):
    h = jnp.maximum(jnp.dot(x_ref[...], w1[...], preferred_element_type=jnp.float32) + b1[...], 0.0)
    h = jnp.dot(h, w2[...], preferred_element_type=jnp.float32) + b2[...]
    o_ref[...] = _ln(h, g[...], be[...])


def _edge_enc_body(ea_ref, w1, b1, w2, b2, g, be, wl0, bl0, wl1, bl1, wl2, bl2,
                   o0_ref, o1_ref, o2_ref):
    h = jnp.maximum(jnp.dot(ea_ref[...], w1[...], preferred_element_type=jnp.float32) + b1[...], 0.0)
    h = jnp.dot(h, w2[...], preferred_element_type=jnp.float32) + b2[...]
    ea = _ln(h, g[...], be[...])
    o0_ref[...] = jnp.dot(ea, wl0[...], preferred_element_type=jnp.float32) + bl0[...]
    o1_ref[...] = jnp.dot(ea, wl1[...], preferred_element_type=jnp.float32) + bl1[...]
    o2_ref[...] = jnp.dot(ea, wl2[...], preferred_element_type=jnp.float32) + bl2[...]


def _node_update_body(x_ref, agg_ref, wm1, bm1, wm2, bm2, g, be, o_ref):
    h = x_ref[...] + agg_ref[0] + agg_ref[1]
    h = jnp.maximum(jnp.dot(h, wm1[...], preferred_element_type=jnp.float32) + bm1[...], 0.0)
    h = jnp.dot(h, wm2[...], preferred_element_type=jnp.float32) + bm2[...]
    o_ref[...] = jnp.maximum(_ln(h, g[...], be[...]), 0.0)


def _heads_body(x_ref, w1, b1, w2, b2, o_ref):
    h = jnp.maximum(jnp.dot(x_ref[...], w1[...], preferred_element_type=jnp.float32) + b1[...], 0.0)
    o = jnp.dot(h, w2[...], preferred_element_type=jnp.float32) + b2[...]
    sp = jnp.maximum(o, 0.0) + jnp.log(1.0 + jnp.exp(-jnp.abs(o)))
    col = lax.broadcasted_iota(jnp.int32, o.shape, 1)
    o_ref[...] = jnp.where(col == 4, sp, o)


def _sc_layer_body(x_hbm, lin_hbm, src_hbm, dst_hbm, out_hbm,
                   srcv, dstv, linA, linB, rowsA, rowsB, aggsh, semA, semB, semS):
    cid = lax.axis_index("c")
    sid = lax.axis_index("s")
    wid = cid * NS + sid

    def zrow(i, carry):
        for k in range(H // 16):
            rowsA[i, pl.ds(k * 16, 16)] = jnp.zeros((16,), jnp.float32)
        return carry
    lax.fori_loop(0, C, zrow, 0)
    for k in range(RPS // C):
        pltpu.sync_copy(rowsA, aggsh.at[pl.ds(sid * RPS + k * C, C)])
    plsc.subcore_barrier()

    def compute(rows, lin):
        def crow(i, c2):
            for r in range(2):
                for k in range(H // 16):
                    sl = (2 * i + r, pl.ds(k * 16, 16))
                    rows[sl] = jnp.maximum(rows[sl] + lin[sl], 0.0)
            return c2
        lax.fori_loop(0, C // 2, crow, 0)

    def group(g, carry):
        pltpu.sync_copy(src_hbm.at[wid * NG + g], srcv)
        pltpu.sync_copy(dst_hbm.at[wid * NG + g], dstv)
        base = g * GI * C

        def lin_cp(j, buf, sem):
            return pltpu.make_async_copy(
                lin_hbm.at[wid, pl.ds(base + j * C, C)], buf, sem)

        def gat_cp(j, buf, sem):
            return pltpu.make_async_copy(x_hbm.at[srcv.at[j]], buf, sem)

        lin_cp(0, linA, semA).start()
        gat_cp(0, rowsA, semA).start()

        def pair(k, c1):
            j0 = 2 * k
            j1 = j0 + 1
            lin_cp(j1, linB, semB).start()
            gat_cp(j1, rowsB, semB).start()
            lin_cp(j0, linA, semA).wait()
            gat_cp(j0, rowsA, semA).wait()
            compute(rowsA, linA)
            pltpu.sync_copy(rowsA, aggsh.at[dstv.at[j0]], add=True)
            lin_cp(j0 + 2, linA, semA).start()
            gat_cp(j0 + 2, rowsA, semA).start()
            lin_cp(j1, linB, semB).wait()
            gat_cp(j1, rowsB, semB).wait()
            compute(rowsB, linB)
            pltpu.sync_copy(rowsB, aggsh.at[dstv.at[j1]], add=True)
            return c1
        lax.fori_loop(0, (GI - 1) // 2, pair, 0)

        lin_cp(GI - 1, linA, semA).wait()
        gat_cp(GI - 1, rowsA, semA).wait()
        compute(rowsA, linA)
        pltpu.sync_copy(rowsA, aggsh.at[dstv.at[GI - 1]], add=True)
        return carry
    lax.fori_loop(0, NG, group, 0)

    plsc.subcore_barrier()
    pltpu.sync_copy(aggsh.at[pl.ds(sid * RPS, RPS)],
                    out_hbm.at[cid, pl.ds(sid * RPS, RPS)])


_sc_layer_cache = []


def _get_sc_layer():
    if not _sc_layer_cache:
        _sc_layer_cache.append(functools.partial(
            pl.kernel,
            out_type=jax.ShapeDtypeStruct((NC, NPAD, H), jnp.float32),
            mesh=plsc.VectorSubcoreMesh(core_axis_name="c", subcore_axis_name="s",
                                        num_cores=NC, num_subcores=NS),
            scratch_types=[
                pltpu.VMEM((GI, C), jnp.int32),
                pltpu.VMEM((GI, C), jnp.int32),
                pltpu.VMEM((C, H), jnp.float32),
                pltpu.VMEM((C, H), jnp.float32),
                pltpu.VMEM((C, H), jnp.float32),
                pltpu.VMEM((C, H), jnp.float32),
                pltpu.VMEM_SHARED((NPAD, H), jnp.float32),
                pltpu.SemaphoreType.DMA,
                pltpu.SemaphoreType.DMA,
                pltpu.SemaphoreType.DMA,
            ],
        )(_sc_layer_body))
    return _sc_layer_cache[0]


def _row_specs(nblk, rows, width):
    return pl.BlockSpec((rows, width), lambda i: (i, 0))


def _full(shape):
    return pl.BlockSpec(shape, lambda i: tuple(0 for _ in shape))


def kernel(x, edge_index, edge_attr, ne_W1, ne_b1, ne_W2, ne_b2, ne_g, ne_be, ee_W1, ee_b1, ee_W2, ee_b2, ee_g, ee_be, c0_Wl, c0_bl, c0_Wm1, c0_bm1, c0_Wm2, c0_bm2, c0_g, c0_be, c1_Wl, c1_bl, c1_Wm1, c1_bm1, c1_Wm2, c1_bm2, c1_g, c1_be, c2_Wl, c2_bl, c2_Wm1, c2_bm1, c2_Wm2, c2_bm2, c2_g, c2_be, hu_W1, hu_b1, hu_W2, hu_b2, hs_W1, hs_b1, hs_W2, hs_b2, hf_W1, hf_b1, hf_W2, hf_b2):
    f32 = jnp.float32
    r2 = lambda v: v.reshape(1, -1)

    BN = 1000
    GN = N // BN

    x0 = pl.pallas_call(
        _node_enc_body,
        grid=(GN,),
        in_specs=[_row_specs(GN, BN, D_IN), _full((D_IN, H)), _full((1, H)),
                  _full((H, H)), _full((1, H)), _full((1, H)), _full((1, H))],
        out_specs=_row_specs(GN, BN, H),
        out_shape=jax.ShapeDtypeStruct((N, H), f32),
    )(x, ne_W1, r2(ne_b1), ne_W2, r2(ne_b2), r2(ne_g), r2(ne_be))

    BE = 2000
    GE = E // BE
    lin0, lin1, lin2 = pl.pallas_call(
        _edge_enc_body,
        grid=(GE,),
        in_specs=[_row_specs(GE, BE, D_EDGE), _full((D_EDGE, H)), _full((1, H)),
                  _full((H, H)), _full((1, H)), _full((1, H)), _full((1, H)),
                  _full((H, H)), _full((1, H)), _full((H, H)), _full((1, H)),
                  _full((H, H)), _full((1, H))],
        out_specs=[_row_specs(GE, BE, H)] * 3,
        out_shape=[jax.ShapeDtypeStruct((E, H), f32)] * 3,
    )(edge_attr, ee_W1, r2(ee_b1), ee_W2, r2(ee_b2), r2(ee_g), r2(ee_be),
      c0_Wl, r2(c0_bl), c1_Wl, r2(c1_bl), c2_Wl, r2(c2_bl))

    src3d = edge_index[0].reshape(NW * NG, GI, C)
    dst3d = edge_index[1].reshape(NW * NG, GI, C)

    layer_w = [
        (lin0, c0_Wm1, c0_bm1, c0_Wm2, c0_bm2, c0_g, c0_be),
        (lin1, c1_Wm1, c1_bm1, c1_Wm2, c1_bm2, c1_g, c1_be),
        (lin2, c2_Wm1, c2_bm1, c2_Wm2, c2_bm2, c2_g, c2_be),
    ]

    xi = x0
    sc_layer = _get_sc_layer()
    for lin, wm1, bm1, wm2, bm2, g, be in layer_w:
        parts = sc_layer(xi, lin.reshape(NW, EPW, H), src3d, dst3d)
        parts = parts[:, :N]
        xi = pl.pallas_call(
            _node_update_body,
            grid=(GN,),
            in_specs=[_row_specs(GN, BN, H),
                      pl.BlockSpec((NC, BN, H), lambda i: (0, i, 0)),
                      _full((H, H)), _full((1, H)), _full((H, H)), _full((1, H)),
                      _full((1, H)), _full((1, H))],
            out_specs=_row_specs(GN, BN, H),
            out_shape=jax.ShapeDtypeStruct((N, H), f32),
        )(xi, parts, wm1, r2(bm1), wm2, r2(bm2), r2(g), r2(be))

    W1c = jnp.concatenate([hu_W1, hs_W1, hf_W1], axis=1)
    b1c = jnp.concatenate([hu_b1, hs_b1, hf_b1]).reshape(1, -1)
    W2c = jnp.zeros((192, 8), f32)
    W2c = W2c.at[0:64, 0:3].set(hu_W2)
    W2c = W2c.at[64:128, 3:4].set(hs_W2)
    W2c = W2c.at[128:192, 4:5].set(hf_W2)
    b2c = jnp.zeros((8,), f32)
    b2c = b2c.at[0:3].set(hu_b2).at[3:4].set(hs_b2).at[4:5].set(hf_b2)
    b2c = b2c.reshape(1, -1)

    heads = pl.pallas_call(
        _heads_body,
        grid=(GN,),
        in_specs=[_row_specs(GN, BN, H), _full((H, 192)), _full((1, 192)),
                  _full((192, 8)), _full((1, 8))],
        out_specs=_row_specs(GN, BN, 8),
        out_shape=jax.ShapeDtypeStruct((N, 8), f32),
    )(xi, W1c, b1c, W2c, b2c)

    return {
        "displacement": heads[:, 0:3],
        "stress": heads[:, 3:4],
        "safety_factor": heads[:, 4:5],
    }
